# Initial kernel scaffold; baseline (speedup 1.0000x reference)
import functools

import jax
import jax.numpy as jnp
from jax import lax
from jax.experimental import pallas as pl
from jax.experimental.pallas import tpu as pltpu
from jax.experimental.pallas import tpu_sc as plsc

N = 10000
D = 128
E = 320000

NC = 2
NS = 16
CHUNK = 80
NCHUNK = E // (NC * NS * CHUNK)
ROWS_PER_TILE = N // NS
DEGW = 16

_MESH = plsc.VectorSubcoreMesh(core_axis_name="c", subcore_axis_name="s")


@functools.partial(
    pl.kernel,
    out_type=jax.ShapeDtypeStruct((NC, N, DEGW), jnp.float32),
    mesh=_MESH,
    scratch_types=[
        pltpu.VMEM((NCHUNK, CHUNK), jnp.int32),
        pltpu.VMEM((CHUNK, DEGW), jnp.float32),
        pltpu.VMEM_SHARED((N, DEGW), jnp.float32),
    ],
)
def _sc_degree(dsts_hbm, ones_hbm, zeros_hbm, out_hbm, dst_v, ones_v, accum):
    cid = lax.axis_index("c")
    sid = lax.axis_index("s")
    rbase = sid * ROWS_PER_TILE
    pltpu.sync_copy(zeros_hbm.at[pl.ds(rbase, ROWS_PER_TILE)],
                    accum.at[pl.ds(rbase, ROWS_PER_TILE)])
    pltpu.sync_copy(dsts_hbm.at[cid, sid], dst_v)
    pltpu.sync_copy(ones_hbm, ones_v)
    plsc.subcore_barrier()

    def body(j, carry):
        pltpu.sync_copy(ones_v, accum.at[dst_v.at[j]], add=True)
        return carry

    lax.fori_loop(0, NCHUNK, body, 0)
    plsc.subcore_barrier()
    pltpu.sync_copy(accum.at[pl.ds(rbase, ROWS_PER_TILE)],
                    out_hbm.at[cid, pl.ds(rbase, ROWS_PER_TILE)])


@functools.partial(
    pl.kernel,
    out_type=jax.ShapeDtypeStruct((NC, N, D), jnp.float32),
    mesh=_MESH,
    scratch_types=[
        pltpu.VMEM((NCHUNK, CHUNK), jnp.int32),
        pltpu.VMEM((NCHUNK, CHUNK), jnp.int32),
        pltpu.VMEM((CHUNK, D), jnp.float32),
        pltpu.VMEM_SHARED((N, D), jnp.float32),
    ],
)
def _sc_aggregate(y_hbm, srcs_hbm, dsts_hbm, zeros_hbm, out_hbm,
                  src_v, dst_v, rows_v, accum):
    cid = lax.axis_index("c")
    sid = lax.axis_index("s")
    rbase = sid * ROWS_PER_TILE
    pltpu.sync_copy(zeros_hbm.at[pl.ds(rbase, ROWS_PER_TILE)],
                    accum.at[pl.ds(rbase, ROWS_PER_TILE)])
    pltpu.sync_copy(srcs_hbm.at[cid, sid], src_v)
    pltpu.sync_copy(dsts_hbm.at[cid, sid], dst_v)
    plsc.subcore_barrier()

    def body(j, carry):
        pltpu.sync_copy(y_hbm.at[src_v.at[j]], rows_v)
        pltpu.sync_copy(rows_v, accum.at[dst_v.at[j]], add=True)
        return carry

    lax.fori_loop(0, NCHUNK, body, 0)
    plsc.subcore_barrier()
    pltpu.sync_copy(accum.at[pl.ds(rbase, ROWS_PER_TILE)],
                    out_hbm.at[cid, pl.ds(rbase, ROWS_PER_TILE)])


def _tc_a_body(degp_ref, emb_ref, dinv_ref, y1_ref):
    deg = degp_ref[0, :, 0:1] + degp_ref[1, :, 0:1] + 1.0
    dinv = lax.rsqrt(deg)
    dinv_ref[...] = dinv
    y1_ref[...] = dinv * emb_ref[...]


def _tc_b_body(z1p_ref, y1_ref, dinv_ref, w1_ref, w2_ref, b1_ref, y2_ref):
    dinv = dinv_ref[...]
    p1 = dinv * (z1p_ref[0] + z1p_ref[1] + y1_ref[...])
    w12 = jnp.dot(w1_ref[...], w2_ref[...], preferred_element_type=jnp.float32)
    bb = jnp.dot(b1_ref[...], w2_ref[...], preferred_element_type=jnp.float32)
    x2 = jnp.dot(p1, w12, preferred_element_type=jnp.float32) + bb
    y2_ref[...] = dinv * x2


def _tc_c_body(z2p_ref, y2_ref, dinv_ref, b2_ref, gamma_ref, beta_ref, out_ref):
    h = dinv_ref[...] * (z2p_ref[0] + z2p_ref[1] + y2_ref[...]) + b2_ref[...]
    mean = jnp.mean(h, axis=0, keepdims=True)
    var = jnp.mean((h - mean) ** 2, axis=0, keepdims=True)
    out_ref[...] = ((h - mean) * lax.rsqrt(var + 1e-5) * gamma_ref[...]
                    + beta_ref[...])


_tc_a = pl.pallas_call(
    _tc_a_body,
    out_shape=(jax.ShapeDtypeStruct((N, 1), jnp.float32),
               jax.ShapeDtypeStruct((N, D), jnp.float32)),
)

_tc_b = pl.pallas_call(
    _tc_b_body,
    out_shape=jax.ShapeDtypeStruct((N, D), jnp.float32),
)

_tc_c = pl.pallas_call(
    _tc_c_body,
    out_shape=jax.ShapeDtypeStruct((N, D), jnp.float32),
)


def kernel(emb, edge_index, W1, b1, W2, b2, gamma, beta):
    src = edge_index[0].reshape(NC, NS, NCHUNK, CHUNK)
    dst = edge_index[1].reshape(NC, NS, NCHUNK, CHUNK)
    zeros_nd = jnp.zeros((N, D), jnp.float32)
    zeros_ndeg = jnp.zeros((N, DEGW), jnp.float32)
    ones_deg = jnp.ones((CHUNK, DEGW), jnp.float32)

    degp = _sc_degree(dst, ones_deg, zeros_ndeg)
    dinv, y1 = _tc_a(degp, emb)
    z1p = _sc_aggregate(y1, src, dst, zeros_nd)
    y2 = _tc_b(z1p, y1, dinv, W1, W2, b1.reshape(1, 2 * D))
    z2p = _sc_aggregate(y2, src, dst, zeros_nd)
    out = _tc_c(z2p, y2, dinv, b2.reshape(1, D),
                gamma.reshape(1, D), beta.reshape(1, D))
    return out


# speedup vs baseline: 20.5817x; 20.5817x over previous
import functools

import jax
import jax.numpy as jnp
from jax import lax
from jax.experimental import pallas as pl
from jax.experimental.pallas import tpu as pltpu
from jax.experimental.pallas import tpu_sc as plsc

N = 10000
D = 128
E = 320000

NC = 2
NS = 16
CHUNK = 80
NCHUNK = E // (NC * NS * CHUNK)
NPAD = 10240
ROWS_PER_TILE = NPAD // NS
DEGW = 16

_MESH = plsc.VectorSubcoreMesh(core_axis_name="c", subcore_axis_name="s")


@functools.partial(
    pl.kernel,
    out_type=jax.ShapeDtypeStruct((NC, NPAD, D), jnp.float32),
    mesh=_MESH,
    scratch_types=[
        pltpu.VMEM((NCHUNK, CHUNK), jnp.int32),
        pltpu.VMEM((CHUNK, D), jnp.float32),
        pltpu.VMEM_SHARED((NPAD, D), jnp.float32),
    ],
)
def _sc_degree(dsts_hbm, ones_hbm, zeros_hbm, out_hbm, dst_v, ones_v, accum):
    cid = lax.axis_index("c")
    sid = lax.axis_index("s")
    rbase = sid * ROWS_PER_TILE
    pltpu.sync_copy(zeros_hbm.at[pl.ds(rbase, ROWS_PER_TILE)],
                    accum.at[pl.ds(rbase, ROWS_PER_TILE)])
    pltpu.sync_copy(dsts_hbm.at[cid, sid], dst_v)
    pltpu.sync_copy(ones_hbm, ones_v)
    plsc.subcore_barrier()

    def body(j, carry):
        pltpu.sync_copy(ones_v, accum.at[dst_v.at[j]], add=True)
        return carry

    lax.fori_loop(0, NCHUNK, body, 0)
    plsc.subcore_barrier()
    pltpu.sync_copy(accum.at[pl.ds(rbase, ROWS_PER_TILE)],
                    out_hbm.at[cid, pl.ds(rbase, ROWS_PER_TILE)])


@functools.partial(
    pl.kernel,
    out_type=jax.ShapeDtypeStruct((NC, NPAD, D), jnp.float32),
    mesh=_MESH,
    scratch_types=[
        pltpu.VMEM((NCHUNK, CHUNK), jnp.int32),
        pltpu.VMEM((NCHUNK, CHUNK), jnp.int32),
        pltpu.VMEM((CHUNK, D), jnp.float32),
        pltpu.VMEM_SHARED((NPAD, D), jnp.float32),
    ],
)
def _sc_aggregate(y_hbm, srcs_hbm, dsts_hbm, zeros_hbm, out_hbm,
                  src_v, dst_v, rows_v, accum):
    cid = lax.axis_index("c")
    sid = lax.axis_index("s")
    rbase = sid * ROWS_PER_TILE
    pltpu.sync_copy(zeros_hbm.at[pl.ds(rbase, ROWS_PER_TILE)],
                    accum.at[pl.ds(rbase, ROWS_PER_TILE)])
    pltpu.sync_copy(srcs_hbm.at[cid, sid], src_v)
    pltpu.sync_copy(dsts_hbm.at[cid, sid], dst_v)
    plsc.subcore_barrier()

    def body(j, carry):
        pltpu.sync_copy(y_hbm.at[src_v.at[j]], rows_v)
        pltpu.sync_copy(rows_v, accum.at[dst_v.at[j]], add=True)
        return carry

    lax.fori_loop(0, NCHUNK, body, 0)
    plsc.subcore_barrier()
    pltpu.sync_copy(accum.at[pl.ds(rbase, ROWS_PER_TILE)],
                    out_hbm.at[cid, pl.ds(rbase, ROWS_PER_TILE)])


def _tc_a_body(degp_ref, emb_ref, dinv_ref, y1_ref):
    deg = degp_ref[0, :N, 0:1] + degp_ref[1, :N, 0:1] + 1.0
    dinv = lax.rsqrt(deg)
    dinv_ref[...] = dinv
    y1_ref[...] = dinv * emb_ref[...]


def _tc_b_body(z1p_ref, y1_ref, dinv_ref, w1_ref, w2_ref, b1_ref, y2_ref):
    dinv = dinv_ref[...]
    p1 = dinv * (z1p_ref[0, :N] + z1p_ref[1, :N] + y1_ref[...])
    w12 = jnp.dot(w1_ref[...], w2_ref[...], preferred_element_type=jnp.float32)
    bb = jnp.dot(b1_ref[...], w2_ref[...], preferred_element_type=jnp.float32)
    x2 = jnp.dot(p1, w12, preferred_element_type=jnp.float32) + bb
    y2_ref[...] = dinv * x2


def _tc_c_body(z2p_ref, y2_ref, dinv_ref, b2_ref, gamma_ref, beta_ref, out_ref):
    h = dinv_ref[...] * (z2p_ref[0, :N] + z2p_ref[1, :N] + y2_ref[...]) + b2_ref[...]
    mean = jnp.mean(h, axis=0, keepdims=True)
    var = jnp.mean((h - mean) ** 2, axis=0, keepdims=True)
    out_ref[...] = ((h - mean) * lax.rsqrt(var + 1e-5) * gamma_ref[...]
                    + beta_ref[...])


_tc_a = pl.pallas_call(
    _tc_a_body,
    out_shape=(jax.ShapeDtypeStruct((N, 1), jnp.float32),
               jax.ShapeDtypeStruct((N, D), jnp.float32)),
)

_tc_b = pl.pallas_call(
    _tc_b_body,
    out_shape=jax.ShapeDtypeStruct((N, D), jnp.float32),
)

_tc_c = pl.pallas_call(
    _tc_c_body,
    out_shape=jax.ShapeDtypeStruct((N, D), jnp.float32),
)


def kernel(emb, edge_index, W1, b1, W2, b2, gamma, beta):
    src = edge_index[0].reshape(NC, NS, NCHUNK, CHUNK)
    dst = edge_index[1].reshape(NC, NS, NCHUNK, CHUNK)
    zeros_nd = jnp.zeros((NPAD, D), jnp.float32)
    ones_rows = jnp.ones((CHUNK, D), jnp.float32)

    degp = _sc_degree(dst, ones_rows, zeros_nd)
    dinv, y1 = _tc_a(degp, emb)
    z1p = _sc_aggregate(y1, src, dst, zeros_nd)
    y2 = _tc_b(z1p, y1, dinv, W1, W2, b1.reshape(1, 2 * D))
    z2p = _sc_aggregate(y2, src, dst, zeros_nd)
    out = _tc_c(z2p, y2, dinv, b2.reshape(1, D),
                gamma.reshape(1, D), beta.reshape(1, D))
    return out


# speedup vs baseline: 28.2073x; 1.3705x over previous
import functools

import jax
import jax.numpy as jnp
from jax import lax
from jax.experimental import pallas as pl
from jax.experimental.pallas import tpu as pltpu
from jax.experimental.pallas import tpu_sc as plsc

N = 10000
D = 128
E = 320000

NC = 2
NS = 16
NW = NC * NS
CHUNK = 128
EPT_RAW = E // NW
NCHUNK = -(-EPT_RAW // CHUNK)
EPT = NCHUNK * CHUNK
NSENT = EPT - EPT_RAW
NPAD = 10240
ROWS_PER_TILE = NPAD // NS

_MESH = plsc.VectorSubcoreMesh(core_axis_name="c", subcore_axis_name="s")


@functools.partial(
    pl.kernel,
    out_type=jax.ShapeDtypeStruct((NC, NPAD, D), jnp.float32),
    mesh=_MESH,
    scratch_types=[
        pltpu.VMEM((NCHUNK, CHUNK), jnp.int32),
        pltpu.VMEM((CHUNK, D), jnp.float32),
        pltpu.VMEM_SHARED((NPAD, D), jnp.float32),
        pltpu.SemaphoreType.DMA,
        pltpu.SemaphoreType.DMA,
    ],
)
def _sc_degree(dsts_hbm, ones_hbm, zeros_hbm, out_hbm, dst_v, ones_v, accum,
               s0, s1):
    cid = lax.axis_index("c")
    sid = lax.axis_index("s")
    rbase = sid * ROWS_PER_TILE
    pltpu.sync_copy(zeros_hbm.at[pl.ds(rbase, ROWS_PER_TILE)],
                    accum.at[pl.ds(rbase, ROWS_PER_TILE)])
    pltpu.sync_copy(dsts_hbm.at[cid, sid], dst_v)
    pltpu.sync_copy(ones_hbm, ones_v)
    plsc.subcore_barrier()

    def s_start(j, sem):
        pltpu.async_copy(ones_v, accum.at[dst_v.at[j]], sem, add=True)

    def s_wait(j, sem):
        pltpu.make_async_copy(ones_v, accum.at[dst_v.at[j]], sem).wait()

    s_start(0, s0)
    s_start(1, s1)

    def pair(i, carry):
        j0 = 2 + 2 * i
        s_wait(j0 - 2, s0)
        s_start(j0, s0)
        s_wait(j0 - 1, s1)
        s_start(j0 + 1, s1)
        return carry

    lax.fori_loop(0, (NCHUNK - 3) // 2, pair, 0)
    s_wait(NCHUNK - 3, s0)
    s_start(NCHUNK - 1, s0)
    s_wait(NCHUNK - 2, s1)
    s_wait(NCHUNK - 1, s0)
    plsc.subcore_barrier()
    pltpu.sync_copy(accum.at[pl.ds(rbase, ROWS_PER_TILE)],
                    out_hbm.at[cid, pl.ds(rbase, ROWS_PER_TILE)])


@functools.partial(
    pl.kernel,
    out_type=jax.ShapeDtypeStruct((NC, NPAD, D), jnp.float32),
    mesh=_MESH,
    scratch_types=[
        pltpu.VMEM((NCHUNK, CHUNK), jnp.int32),
        pltpu.VMEM((CHUNK,), jnp.int32),
        pltpu.VMEM((CHUNK,), jnp.int32),
        pltpu.VMEM((CHUNK,), jnp.int32),
        pltpu.VMEM((CHUNK,), jnp.int32),
        pltpu.VMEM((CHUNK, D), jnp.float32),
        pltpu.VMEM((CHUNK, D), jnp.float32),
        pltpu.VMEM_SHARED((NPAD, D), jnp.float32),
        pltpu.SemaphoreType.DMA,
        pltpu.SemaphoreType.DMA,
        pltpu.SemaphoreType.DMA,
        pltpu.SemaphoreType.DMA,
    ],
)
def _sc_aggregate(y_hbm, packed_hbm, zeros_hbm, out_hbm,
                  slab, sb0, sb1, db0, db1, rows0, rows1, accum,
                  g0, g1, s0, s1):
    cid = lax.axis_index("c")
    sid = lax.axis_index("s")
    rbase = sid * ROWS_PER_TILE
    pltpu.sync_copy(zeros_hbm.at[pl.ds(rbase, ROWS_PER_TILE)],
                    accum.at[pl.ds(rbase, ROWS_PER_TILE)])
    pltpu.sync_copy(packed_hbm.at[cid, sid], slab)
    plsc.subcore_barrier()

    def unpack(j, sb, db):
        for k in range(CHUNK // 16):
            pk = slab[j, pl.ds(k * 16, 16)]
            sb[pl.ds(k * 16, 16)] = pk & 0xFFFF
            db[pl.ds(k * 16, 16)] = pk >> 16

    def g_start(sb, rows, sem):
        pltpu.async_copy(y_hbm.at[sb], rows, sem)

    def g_wait(sb, rows, sem):
        pltpu.make_async_copy(y_hbm.at[sb], rows, sem).wait()

    def s_start(rows, db, sem):
        pltpu.async_copy(rows, accum.at[db], sem, add=True)

    def s_wait(rows, db, sem):
        pltpu.make_async_copy(rows, accum.at[db], sem).wait()

    def body(j, has_next, sb, db, rows, gsem, ssem, nsb, ndb, nrows,
             ngsem, nssem, first=False):
        g_wait(sb, rows, gsem)
        s_start(rows, db, ssem)
        if not first:
            s_wait(nrows, ndb, nssem)
        if has_next:
            unpack(j + 1, nsb, ndb)
            g_start(nsb, nrows, ngsem)

    unpack(0, sb0, db0)
    g_start(sb0, rows0, g0)
    body(0, True, sb0, db0, rows0, g0, s0, sb1, db1, rows1, g1, s1,
         first=True)

    def pair(i, carry):
        j1 = 1 + 2 * i
        body(j1, True, sb1, db1, rows1, g1, s1, sb0, db0, rows0, g0, s0)
        body(j1 + 1, True, sb0, db0, rows0, g0, s0, sb1, db1, rows1, g1, s1)
        return carry

    lax.fori_loop(0, (NCHUNK - 3) // 2, pair, 0)
    body(NCHUNK - 2, True, sb1, db1, rows1, g1, s1,
         sb0, db0, rows0, g0, s0)
    body(NCHUNK - 1, False, sb0, db0, rows0, g0, s0,
         sb1, db1, rows1, g1, s1)
    s_wait(rows0, db0, s0)
    plsc.subcore_barrier()
    pltpu.sync_copy(accum.at[pl.ds(rbase, ROWS_PER_TILE)],
                    out_hbm.at[cid, pl.ds(rbase, ROWS_PER_TILE)])


def _tc_a_body(degp_ref, emb_ref, dinv_ref, y1_ref):
    deg = degp_ref[0, :N, 0:1] + degp_ref[1, :N, 0:1] + 1.0
    dinv = lax.rsqrt(deg)
    dinv_ref[...] = dinv
    y1_ref[:N] = dinv * emb_ref[...]
    y1_ref[N:] = jnp.zeros((NPAD - N, D), jnp.float32)


def _tc_b_body(z1p_ref, y1_ref, dinv_ref, w1_ref, w2_ref, b1_ref, y2_ref):
    dinv = dinv_ref[...]
    p1 = dinv * (z1p_ref[0, :N] + z1p_ref[1, :N] + y1_ref[:N])
    w12 = jnp.dot(w1_ref[...], w2_ref[...], preferred_element_type=jnp.float32)
    bb = jnp.dot(b1_ref[...], w2_ref[...], preferred_element_type=jnp.float32)
    x2 = jnp.dot(p1, w12, preferred_element_type=jnp.float32) + bb
    y2_ref[:N] = dinv * x2
    y2_ref[N:] = jnp.zeros((NPAD - N, D), jnp.float32)


def _tc_c_body(z2p_ref, y2_ref, dinv_ref, b2_ref, gamma_ref, beta_ref, out_ref):
    h = (dinv_ref[...] * (z2p_ref[0, :N] + z2p_ref[1, :N] + y2_ref[:N])
         + b2_ref[...])
    mean = jnp.mean(h, axis=0, keepdims=True)
    var = jnp.mean((h - mean) ** 2, axis=0, keepdims=True)
    out_ref[...] = ((h - mean) * lax.rsqrt(var + 1e-5) * gamma_ref[...]
                    + beta_ref[...])


_tc_a = pl.pallas_call(
    _tc_a_body,
    out_shape=(jax.ShapeDtypeStruct((N, 1), jnp.float32),
               jax.ShapeDtypeStruct((NPAD, D), jnp.float32)),
)

_tc_b = pl.pallas_call(
    _tc_b_body,
    out_shape=jax.ShapeDtypeStruct((NPAD, D), jnp.float32),
)

_tc_c = pl.pallas_call(
    _tc_c_body,
    out_shape=jax.ShapeDtypeStruct((N, D), jnp.float32),
)


def kernel(emb, edge_index, W1, b1, W2, b2, gamma, beta):
    src_t = edge_index[0].reshape(NW, EPT_RAW)
    dst_t = edge_index[1].reshape(NW, EPT_RAW)
    sent = jnp.broadcast_to(jnp.arange(NSENT, dtype=jnp.int32) + N,
                            (NW, NSENT))
    src_p = jnp.concatenate([src_t, sent], axis=1)
    dst_p = jnp.concatenate([dst_t, sent], axis=1)
    packed = (src_p | (dst_p << 16)).reshape(NC, NS, NCHUNK, CHUNK)
    dst4 = dst_p.reshape(NC, NS, NCHUNK, CHUNK)
    zeros_nd = jnp.zeros((NPAD, D), jnp.float32)
    ones_rows = jnp.ones((CHUNK, D), jnp.float32)

    degp = _sc_degree(dst4, ones_rows, zeros_nd)
    dinv, y1 = _tc_a(degp, emb)
    z1p = _sc_aggregate(y1, packed, zeros_nd)
    y2 = _tc_b(z1p, y1, dinv, W1, W2, b1.reshape(1, 2 * D))
    z2p = _sc_aggregate(y2, packed, zeros_nd)
    out = _tc_c(z2p, y2, dinv, b2.reshape(1, D),
                gamma.reshape(1, D), beta.reshape(1, D))
    return out


# speedup vs baseline: 32.7836x; 1.1622x over previous
import functools

import jax
import jax.numpy as jnp
from jax import lax
from jax.experimental import pallas as pl
from jax.experimental.pallas import tpu as pltpu
from jax.experimental.pallas import tpu_sc as plsc

N = 10000
D = 128
E = 320000

NC = 2
NS = 16
NW = NC * NS
CHUNK = 128
EPT_RAW = E // NW
NCHUNK = -(-EPT_RAW // CHUNK)
EPT = NCHUNK * CHUNK
NSENT = EPT - EPT_RAW
NPAD = 10240
ROWS_PER_TILE = NPAD // NS

_MESH = plsc.VectorSubcoreMesh(core_axis_name="c", subcore_axis_name="s")


@functools.partial(
    pl.kernel,
    out_type=jax.ShapeDtypeStruct((NC * NPAD,), jnp.float32),
    mesh=_MESH,
    scratch_types=[
        pltpu.VMEM((NCHUNK, CHUNK), jnp.int32),
        pltpu.VMEM((CHUNK,), jnp.float32),
        pltpu.VMEM_SHARED((NPAD,), jnp.float32),
        pltpu.SemaphoreType.DMA,
        pltpu.SemaphoreType.DMA,
    ],
)
def _sc_degree(dsts_hbm, ones_hbm, zeros_hbm, out_hbm, dst_v, ones_v, accum,
               s0, s1):
    cid = lax.axis_index("c")
    sid = lax.axis_index("s")
    rbase = sid * (NPAD // NS)
    pltpu.sync_copy(zeros_hbm.at[pl.ds(rbase, NPAD // NS)],
                    accum.at[pl.ds(rbase, NPAD // NS)])
    pltpu.sync_copy(dsts_hbm.at[cid, sid], dst_v)
    pltpu.sync_copy(ones_hbm, ones_v)
    plsc.subcore_barrier()

    def s_start(j, sem):
        pltpu.async_copy(ones_v, accum.at[dst_v.at[j]], sem, add=True)

    def s_wait(j, sem):
        pltpu.make_async_copy(ones_v, accum.at[dst_v.at[j]], sem).wait()

    s_start(0, s0)
    s_start(1, s1)

    def pair(i, carry):
        j0 = 2 + 2 * i
        s_wait(j0 - 2, s0)
        s_start(j0, s0)
        s_wait(j0 - 1, s1)
        s_start(j0 + 1, s1)
        return carry

    lax.fori_loop(0, (NCHUNK - 3) // 2, pair, 0)
    s_wait(NCHUNK - 3, s0)
    s_start(NCHUNK - 1, s0)
    s_wait(NCHUNK - 2, s1)
    s_wait(NCHUNK - 1, s0)
    plsc.subcore_barrier()
    pltpu.sync_copy(accum.at[pl.ds(rbase, NPAD // NS)],
                    out_hbm.at[pl.ds(cid * NPAD + rbase, NPAD // NS)])


@functools.partial(
    pl.kernel,
    out_type=jax.ShapeDtypeStruct((NC, NPAD, D), jnp.float32),
    mesh=_MESH,
    scratch_types=[
        pltpu.VMEM((NCHUNK, CHUNK), jnp.int32),
        pltpu.VMEM((CHUNK,), jnp.int32),
        pltpu.VMEM((CHUNK,), jnp.int32),
        pltpu.VMEM((CHUNK,), jnp.int32),
        pltpu.VMEM((CHUNK,), jnp.int32),
        pltpu.VMEM((CHUNK, D), jnp.float32),
        pltpu.VMEM((CHUNK, D), jnp.float32),
        pltpu.VMEM_SHARED((NPAD, D), jnp.float32),
        pltpu.SemaphoreType.DMA,
        pltpu.SemaphoreType.DMA,
        pltpu.SemaphoreType.DMA,
        pltpu.SemaphoreType.DMA,
    ],
)
def _sc_aggregate(y_hbm, packed_hbm, zeros_hbm, out_hbm,
                  slab, sb0, sb1, db0, db1, rows0, rows1, accum,
                  g0, g1, s0, s1):
    cid = lax.axis_index("c")
    sid = lax.axis_index("s")
    rbase = sid * ROWS_PER_TILE
    pltpu.sync_copy(zeros_hbm.at[pl.ds(rbase, ROWS_PER_TILE)],
                    accum.at[pl.ds(rbase, ROWS_PER_TILE)])
    pltpu.sync_copy(packed_hbm.at[cid, sid], slab)
    plsc.subcore_barrier()

    def unpack(j, sb, db):
        for k in range(CHUNK // 16):
            pk = slab[j, pl.ds(k * 16, 16)]
            sb[pl.ds(k * 16, 16)] = pk & 0xFFFF
            db[pl.ds(k * 16, 16)] = pk >> 16

    def g_start(sb, rows, sem):
        pltpu.async_copy(y_hbm.at[sb], rows, sem)

    def g_wait(sb, rows, sem):
        pltpu.make_async_copy(y_hbm.at[sb], rows, sem).wait()

    def s_start(rows, db, sem):
        pltpu.async_copy(rows, accum.at[db], sem, add=True)

    def s_wait(rows, db, sem):
        pltpu.make_async_copy(rows, accum.at[db], sem).wait()

    def body(j, has_next, sb, db, rows, gsem, ssem, nsb, ndb, nrows,
             ngsem, nssem, first=False):
        g_wait(sb, rows, gsem)
        s_start(rows, db, ssem)
        if not first:
            s_wait(nrows, ndb, nssem)
        if has_next:
            unpack(j + 1, nsb, ndb)
            g_start(nsb, nrows, ngsem)

    unpack(0, sb0, db0)
    g_start(sb0, rows0, g0)
    body(0, True, sb0, db0, rows0, g0, s0, sb1, db1, rows1, g1, s1,
         first=True)

    def pair(i, carry):
        j1 = 1 + 2 * i
        body(j1, True, sb1, db1, rows1, g1, s1, sb0, db0, rows0, g0, s0)
        body(j1 + 1, True, sb0, db0, rows0, g0, s0, sb1, db1, rows1, g1, s1)
        return carry

    lax.fori_loop(0, (NCHUNK - 3) // 2, pair, 0)
    body(NCHUNK - 2, True, sb1, db1, rows1, g1, s1,
         sb0, db0, rows0, g0, s0)
    body(NCHUNK - 1, False, sb0, db0, rows0, g0, s0,
         sb1, db1, rows1, g1, s1)
    s_wait(rows0, db0, s0)
    plsc.subcore_barrier()
    pltpu.sync_copy(accum.at[pl.ds(rbase, ROWS_PER_TILE)],
                    out_hbm.at[cid, pl.ds(rbase, ROWS_PER_TILE)])


def _tc_a_body(degp_ref, emb_ref, dinv_ref, y1_ref):
    deg = degp_ref[:N, 0:1] + degp_ref[:N, 1:2] + 1.0
    dinv = lax.rsqrt(deg)
    dinv_ref[...] = dinv
    y1_ref[:N] = dinv * emb_ref[...]
    y1_ref[N:] = jnp.zeros((NPAD - N, D), jnp.float32)


def _tc_b_body(z1p_ref, y1_ref, dinv_ref, w1_ref, w2_ref, b1_ref, y2_ref):
    dinv = dinv_ref[...]
    p1 = dinv * (z1p_ref[0, :N] + z1p_ref[1, :N] + y1_ref[:N])
    w12 = jnp.dot(w1_ref[...], w2_ref[...], preferred_element_type=jnp.float32)
    bb = jnp.dot(b1_ref[...], w2_ref[...], preferred_element_type=jnp.float32)
    x2 = jnp.dot(p1, w12, preferred_element_type=jnp.float32) + bb
    y2_ref[:N] = dinv * x2
    y2_ref[N:] = jnp.zeros((NPAD - N, D), jnp.float32)


def _tc_c_body(z2p_ref, y2_ref, dinv_ref, b2_ref, gamma_ref, beta_ref, out_ref):
    h = (dinv_ref[...] * (z2p_ref[0, :N] + z2p_ref[1, :N] + y2_ref[:N])
         + b2_ref[...])
    mean = jnp.mean(h, axis=0, keepdims=True)
    var = jnp.mean((h - mean) ** 2, axis=0, keepdims=True)
    out_ref[...] = ((h - mean) * lax.rsqrt(var + 1e-5) * gamma_ref[...]
                    + beta_ref[...])


_tc_a = pl.pallas_call(
    _tc_a_body,
    out_shape=(jax.ShapeDtypeStruct((N, 1), jnp.float32),
               jax.ShapeDtypeStruct((NPAD, D), jnp.float32)),
)

_tc_b = pl.pallas_call(
    _tc_b_body,
    out_shape=jax.ShapeDtypeStruct((NPAD, D), jnp.float32),
)

_tc_c = pl.pallas_call(
    _tc_c_body,
    out_shape=jax.ShapeDtypeStruct((N, D), jnp.float32),
)


def kernel(emb, edge_index, W1, b1, W2, b2, gamma, beta):
    src_t = edge_index[0].reshape(NW, EPT_RAW)
    dst_t = edge_index[1].reshape(NW, EPT_RAW)
    sent = jnp.broadcast_to(jnp.arange(NSENT, dtype=jnp.int32) + N,
                            (NW, NSENT))
    src_p = jnp.concatenate([src_t, sent], axis=1)
    dst_p = jnp.concatenate([dst_t, sent], axis=1)
    packed = (src_p | (dst_p << 16)).reshape(NC, NS, NCHUNK, CHUNK)
    dst4 = dst_p.reshape(NC, NS, NCHUNK, CHUNK)
    zeros_nd = jnp.zeros((NPAD, D), jnp.float32)
    zeros_1d = jnp.zeros((NPAD,), jnp.float32)
    ones_1d = jnp.ones((CHUNK,), jnp.float32)

    degp = _sc_degree(dst4, ones_1d, zeros_1d)
    degp_t = degp.reshape(NC, NPAD).T
    dinv, y1 = _tc_a(degp_t, emb)
    z1p = _sc_aggregate(y1, packed, zeros_nd)
    y2 = _tc_b(z1p, y1, dinv, W1, W2, b1.reshape(1, 2 * D))
    z2p = _sc_aggregate(y2, packed, zeros_nd)
    out = _tc_c(z2p, y2, dinv, b2.reshape(1, D),
                gamma.reshape(1, D), beta.reshape(1, D))
    return out
